# baseline (speedup 1.0000x reference)
import functools

import jax
import jax.numpy as jnp
from jax import lax
from jax.experimental import pallas as pl
from jax.experimental.pallas import tpu as pltpu
from jax.experimental.pallas import tpu_sc as plsc

_B, _D = 16384, 4096
_NW = 32
_ROWS_PER_W = _B // _NW
_CHUNK = 2
_CW = _CHUNK * _D
_NCHUNK = _ROWS_PER_W // _CHUNK
_NSLOT = 8
_LEAD = 4
_IDX_PAD = 416
_NJ = _IDX_PAD // 16


def _sc_body(x_hbm, idx_hbm, out_hbm, idx_v, *rest):
    bufs = rest[:_NSLOT]
    in_sems = rest[_NSLOT:2 * _NSLOT]
    out_sems = rest[2 * _NSLOT:3 * _NSLOT]
    c = lax.axis_index("c")
    s = lax.axis_index("s")
    wid = s * 2 + c
    base = wid * _ROWS_PER_W * _D
    pltpu.sync_copy(idx_hbm, idx_v)
    zeros = jnp.zeros((16,), jnp.float32)

    def in_slice(kk):
        return x_hbm.at[pl.ds(base + kk * _CW, _CW)]

    def out_slice(kk):
        return out_hbm.at[pl.ds(base + kk * _CW, _CW)]

    for kk in range(_NSLOT - _LEAD):
        pltpu.make_async_copy(in_slice(kk), bufs[kk], in_sems[kk]).start()

    def round_body(g, carry):
        for b in range(_NSLOT):
            kk = g * _NSLOT + b
            sb = (b - _LEAD) % _NSLOT

            @pl.when(kk >= _LEAD)
            def _():
                pltpu.make_async_copy(
                    in_slice(kk - _LEAD), bufs[sb], out_sems[sb]).wait()

            @pl.when(kk + (_NSLOT - _LEAD) < _NCHUNK)
            def _():
                pltpu.make_async_copy(
                    in_slice(kk + (_NSLOT - _LEAD)), bufs[sb],
                    in_sems[sb]).start()

            pltpu.make_async_copy(in_slice(kk), bufs[b], in_sems[b]).wait()
            for r in range(_CHUNK):
                for j in range(_NJ):
                    colv = idx_v[pl.ds(j * 16, 16)]
                    plsc.store_scatter(bufs[b], [colv + r * _D], zeros)
            pltpu.make_async_copy(bufs[b], out_slice(kk), out_sems[b]).start()
        return carry

    lax.fori_loop(0, _NCHUNK // _NSLOT, round_body, 0)

    for kk in range(_NCHUNK - _LEAD, _NCHUNK):
        b = kk % _NSLOT
        pltpu.make_async_copy(in_slice(kk), bufs[b], out_sems[b]).wait()


def kernel(x, mask_indices):
    idx = mask_indices.astype(jnp.int32)
    n = idx.shape[0]
    idx = jnp.pad(idx, (0, _IDX_PAD - n), mode="edge")

    mesh = plsc.VectorSubcoreMesh(core_axis_name="c", subcore_axis_name="s")
    run = functools.partial(
        pl.kernel,
        mesh=mesh,
        out_type=jax.ShapeDtypeStruct((_B * _D,), jnp.float32),
        scratch_types=(
            [pltpu.VMEM((_IDX_PAD,), jnp.int32)]
            + [pltpu.VMEM((_CW,), jnp.float32) for _ in range(_NSLOT)]
            + [pltpu.SemaphoreType.DMA for _ in range(2 * _NSLOT)]
        ),
        compiler_params=pltpu.CompilerParams(needs_layout_passes=False),
    )(_sc_body)
    return run(x.reshape(_B * _D), idx).reshape(_B, _D)


# speedup vs baseline: 3.5427x; 3.5427x over previous
import functools

import jax
import jax.numpy as jnp
from jax import lax
from jax.experimental import pallas as pl
from jax.experimental.pallas import tpu as pltpu
from jax.experimental.pallas import tpu_sc as plsc

_B, _D = 16384, 4096
_BLOCK_ROWS = 512
_IDX_PAD = 416
_NJ = _IDX_PAD // 16
_FILL = _D // 16


def _sc_mask_body(idx_hbm, mask_hbm, idx_v, mask_v, sem):
    c = lax.axis_index("c")
    s = lax.axis_index("s")
    wid = s * 2 + c

    @pl.when(wid == 0)
    def _():
        pltpu.sync_copy(idx_hbm, idx_v)
        ones = jnp.ones((16,), jnp.float32)
        zeros = jnp.zeros((16,), jnp.float32)

        def fill_body(i, carry):
            mask_v[pl.ds(i * 16, 16)] = ones
            return carry

        lax.fori_loop(0, _FILL, fill_body, 0)

        def j_body(j, carry):
            colv = idx_v[pl.ds(j * 16, 16)]
            plsc.store_scatter(mask_v, [colv], zeros)
            return carry

        lax.fori_loop(0, _NJ, j_body, 0)
        pltpu.sync_copy(mask_v, mask_hbm)


def _tc_body(mask_ref, x_ref, o_ref):
    o_ref[...] = x_ref[...] * mask_ref[...]


def kernel(x, mask_indices):
    idx = mask_indices.astype(jnp.int32)
    n = idx.shape[0]
    idx = jnp.pad(idx, (0, _IDX_PAD - n), mode="edge")

    mesh = plsc.VectorSubcoreMesh(core_axis_name="c", subcore_axis_name="s")
    build_mask = functools.partial(
        pl.kernel,
        mesh=mesh,
        out_type=jax.ShapeDtypeStruct((_D,), jnp.float32),
        scratch_types=[
            pltpu.VMEM((_IDX_PAD,), jnp.int32),
            pltpu.VMEM((_D,), jnp.float32),
            pltpu.SemaphoreType.DMA,
        ],
        compiler_params=pltpu.CompilerParams(needs_layout_passes=False),
    )(_sc_mask_body)
    mask = build_mask(idx).reshape(1, _D)

    grid = (_B // _BLOCK_ROWS,)
    return pl.pallas_call(
        _tc_body,
        grid=grid,
        in_specs=[
            pl.BlockSpec((1, _D), lambda i: (0, 0)),
            pl.BlockSpec((_BLOCK_ROWS, _D), lambda i: (i, 0)),
        ],
        out_specs=pl.BlockSpec((_BLOCK_ROWS, _D), lambda i: (i, 0)),
        out_shape=jax.ShapeDtypeStruct((_B, _D), jnp.float32),
        compiler_params=pltpu.CompilerParams(
            dimension_semantics=("arbitrary",),
        ),
    )(mask, x)
